# baseline (speedup 1.0000x reference)
import functools
import jax
import jax.numpy as jnp
from jax import lax
from jax.experimental import pallas as pl
from jax.experimental.pallas import tpu as pltpu
from jax.experimental.pallas import tpu_sc as plsc

N_NODES = 10000
N_EDGES = 320000
D = 128
H = 128

NODE_BLK = 512
EDGE_BLK = 512
NP = 10240

NC = 2
NS = 16
NW = NC * NS
NPW = NP // NW
EW = N_EDGES // NW

FL = 2048
CBUF = 8192
CAPW = N_EDGES + FL
CH = 3200
CG = 80
EID_BITS = 19
EID_MASK = (1 << EID_BITS) - 1

SCAP = 24576
NG = SCAP // 16
NH = 336

_INTERPRET = False

_mesh = plsc.VectorSubcoreMesh(
    core_axis_name="c", subcore_axis_name="s", num_cores=NC, num_subcores=NS)


def _mo(v, n=8):
  return pl.multiple_of(v, n)


def _wid():
  return lax.axis_index("s") * NC + lax.axis_index("c")


def _linear_body(x_ref, wt_ref, b_ref, o_ref, *, act):
  y = jnp.dot(x_ref[...], wt_ref[...], preferred_element_type=jnp.float32)
  y = y + b_ref[...]
  if act:
    y = jnp.maximum(y, 0.0)
  o_ref[...] = y


def _linear(x, wt, b, act):
  n = x.shape[0]
  grid = n // NODE_BLK
  return pl.pallas_call(
      functools.partial(_linear_body, act=act),
      grid=(grid,),
      in_specs=[
          pl.BlockSpec((NODE_BLK, D), lambda i: (i, 0)),
          pl.BlockSpec((D, D), lambda i: (0, 0)),
          pl.BlockSpec((1, D), lambda i: (0, 0)),
      ],
      out_specs=pl.BlockSpec((NODE_BLK, D), lambda i: (i, 0)),
      out_shape=jax.ShapeDtypeStruct((n, D), jnp.float32),
      interpret=_INTERPRET,
  )(x, wt, b)


def _node_body(h_ref, w1it_ref, w1jt_ref, b1_ref, w2t_ref, b2_ref,
               a_ref, b_ref, s_ref):
  h = h_ref[...]
  a = jnp.dot(h, w1it_ref[...], preferred_element_type=jnp.float32)
  b = jnp.dot(h, w1jt_ref[...], preferred_element_type=jnp.float32)
  a_ref[...] = a
  b_ref[...] = b
  t = jnp.maximum(a + b + b1_ref[...], 0.0)
  s_ref[...] = jnp.dot(t, w2t_ref[...],
                       preferred_element_type=jnp.float32) + b2_ref[...]


def _node_kernel(h, w1it, w1jt, b1, w2t, b2):
  n = h.shape[0]
  grid = n // NODE_BLK
  out = jax.ShapeDtypeStruct((n, D), jnp.float32)
  return pl.pallas_call(
      _node_body,
      grid=(grid,),
      in_specs=[
          pl.BlockSpec((NODE_BLK, D), lambda i: (i, 0)),
          pl.BlockSpec((D, D), lambda i: (0, 0)),
          pl.BlockSpec((D, D), lambda i: (0, 0)),
          pl.BlockSpec((1, D), lambda i: (0, 0)),
          pl.BlockSpec((D, D), lambda i: (0, 0)),
          pl.BlockSpec((1, D), lambda i: (0, 0)),
      ],
      out_specs=[pl.BlockSpec((NODE_BLK, D), lambda i: (i, 0))] * 3,
      out_shape=[out, out, out],
      interpret=_INTERPRET,
  )(h, w1it, w1jt, b1, w2t, b2)


def _edge_body(g_ref, ea_ref, w1et_ref, b1_ref, w2t_ref, b2_ref, m_ref):
  t = g_ref[...] + jnp.dot(ea_ref[...], w1et_ref[...],
                           preferred_element_type=jnp.float32) + b1_ref[...]
  t = jnp.maximum(t, 0.0)
  m_ref[...] = jnp.dot(t, w2t_ref[...],
                       preferred_element_type=jnp.float32) + b2_ref[...]


def _edge_kernel(g, eap, w1et, b1, w2t, b2):
  e = g.shape[0]
  grid = e // EDGE_BLK
  return pl.pallas_call(
      _edge_body,
      grid=(grid,),
      in_specs=[
          pl.BlockSpec((EDGE_BLK, D), lambda i: (i, 0)),
          pl.BlockSpec((EDGE_BLK, 8), lambda i: (i, 0)),
          pl.BlockSpec((8, D), lambda i: (0, 0)),
          pl.BlockSpec((1, D), lambda i: (0, 0)),
          pl.BlockSpec((D, D), lambda i: (0, 0)),
          pl.BlockSpec((1, D), lambda i: (0, 0)),
      ],
      out_specs=pl.BlockSpec((EDGE_BLK, D), lambda i: (i, 0)),
      out_shape=jax.ShapeDtypeStruct((e, D), jnp.float32),
      interpret=_INTERPRET,
  )(g, eap, w1et, b1, w2t, b2)


@functools.partial(
    pl.kernel,
    out_type=[jax.ShapeDtypeStruct((NW * CAPW,), jnp.int32),
              jax.ShapeDtypeStruct((NW * CAPW,), jnp.int32),
              jax.ShapeDtypeStruct((NW * 16,), jnp.int32),
              jax.ShapeDtypeStruct((NW * SCAP,), jnp.int32),
              jax.ShapeDtypeStruct((NW * NG,), jnp.int32)],
    mesh=_mesh,
    compiler_params=pltpu.CompilerParams(needs_layout_passes=False),
    scratch_types=[
        pltpu.VMEM((CH,), jnp.int32),
        pltpu.VMEM((CBUF,), jnp.int32),
        pltpu.VMEM((CBUF,), jnp.int32),
        pltpu.VMEM((NH,), jnp.int32),
        pltpu.VMEM((NH,), jnp.int32),
        pltpu.VMEM((NH,), jnp.int32),
        pltpu.VMEM((SCAP,), jnp.int32),
        pltpu.VMEM((NG,), jnp.int32),
        pltpu.SMEM((2 * NH,), jnp.int32),
    ],
)
def _sc_bin(dst_hbm, eids_hbm, lds_hbm, cnts_hbm, sort_hbm, gl_hbm,
            dbuf, ebuf, lbuf, histv, laste, bbuf, sbufv, glbuf, sm):
  w = _wid()
  base = w * NPW
  lim = base + NPW
  iota = lax.iota(jnp.int32, 16)
  sent_ld = jnp.full((16,), NPW, jnp.int32)
  zero = jnp.full((16,), 0, jnp.int32)
  ones = jnp.full((16,), 1, jnp.int32)

  def init(i, _):
    ebuf[pl.ds(_mo(i * 16, 16), 16)] = zero
    lbuf[pl.ds(_mo(i * 16, 16), 16)] = sent_ld
    return 0
  lax.fori_loop(0, CBUF // 16, init, 0)

  def init2(i, _):
    sbufv[pl.ds(_mo(i * 16, 16), 16)] = zero
    return 0
  lax.fori_loop(0, SCAP // 16, init2, 0)

  def init3(i, _):
    glbuf[pl.ds(_mo(i * 16, 16), 16)] = sent_ld
    return 0
  lax.fori_loop(0, NG // 16, init3, 0)

  for t in range(NH // 16):
    histv[pl.ds(t * 16, 16)] = zero
    laste[pl.ds(t * 16, 16)] = zero

  def chunk(c, carry):
    p, fl, cnt = carry
    pltpu.sync_copy(dst_hbm.at[pl.ds(_mo(c * CH), CH)], dbuf)

    def step(i, carry2):
      p, cnt = carry2
      v = dbuf[pl.ds(_mo(i * 16, 16), 16)]
      eidv = c * CH + i * 16 + iota
      msk = jnp.logical_and(v >= base, v < lim)
      ldv = v - base
      plsc.store_compressed(ebuf.at[pl.ds(p, 16)], eidv, mask=msk)
      plsc.store_compressed(lbuf.at[pl.ds(p, 16)], ldv, mask=msk)
      plsc.addupdate_scatter(histv, [jnp.where(msk, ldv, NPW)], ones, mask=msk)
      plsc.store_scatter(laste, [jnp.where(msk, ldv, NPW)], eidv, mask=msk)
      pc = jnp.max(plsc.all_reduce_population_count(msk))
      return p + pc, cnt + pc

    p, cnt = lax.fori_loop(0, CH // 16, step, (p, cnt))

    def flush_cond(fc):
      return fc[0] >= FL

    def flush(fc):
      p, fl = fc
      pltpu.sync_copy(ebuf.at[pl.ds(0, FL)],
                      eids_hbm.at[pl.ds(_mo(w * CAPW + fl), FL)])
      pltpu.sync_copy(lbuf.at[pl.ds(0, FL)],
                      lds_hbm.at[pl.ds(_mo(w * CAPW + fl), FL)])

      def shift(i, _):
        ebuf[pl.ds(_mo(i * 16, 16), 16)] = ebuf[pl.ds(_mo(i * 16 + FL, 16), 16)]
        lbuf[pl.ds(_mo(i * 16, 16), 16)] = lbuf[pl.ds(_mo(i * 16 + FL, 16), 16)]
        return 0

      lax.fori_loop(0, (CBUF - FL) // 16, shift, 0)
      return p - FL, fl + FL

    p, fl = lax.while_loop(flush_cond, flush, (p, fl))
    return p, fl, cnt

  p, fl, cnt = lax.fori_loop(0, N_EDGES // CH, chunk, (0, 0, 0))

  def drain_cond(fc):
    return fc[0] > 0

  def drain(fc):
    p, fl = fc
    pltpu.sync_copy(ebuf.at[pl.ds(0, FL)],
                    eids_hbm.at[pl.ds(_mo(w * CAPW + fl), FL)])
    pltpu.sync_copy(lbuf.at[pl.ds(0, FL)],
                    lds_hbm.at[pl.ds(_mo(w * CAPW + fl), FL)])
    return p - FL, fl + FL

  p, fl = lax.while_loop(drain_cond, drain, (p, fl))

  def bases(t, carry):
    h = histv[pl.ds(_mo(t * 16, 16), 16)]
    hp = jnp.bitwise_and(h + 15, ~15)
    cs = plsc.cumsum(hp)
    bbuf[pl.ds(_mo(t * 16, 16), 16)] = cs - hp + carry
    return carry + jnp.max(cs)

  total = lax.fori_loop(0, NH // 16, bases, 0)
  ok = jnp.int32(total <= SCAP)
  ngr = total // 16

  def extract(t, _):
    bv = bbuf[pl.ds(_mo(t * 16, 16), 16)]
    for lane in range(16):
      s = jnp.max(jnp.where(iota == lane, bv, 0))
      sm[t * 16 + lane] = s
      sm[NH + t * 16 + lane] = s
    return 0

  lax.fori_loop(0, NH // 16, extract, 0)

  @pl.when(ok == 1)
  def _sort():
    nch = (cnt + FL - 1) // FL

    def pchunk(c, _):
      pltpu.sync_copy(eids_hbm.at[pl.ds(_mo(w * CAPW + c * FL), FL)],
                      ebuf.at[pl.ds(0, FL)])
      pltpu.sync_copy(lds_hbm.at[pl.ds(_mo(w * CAPW + c * FL), FL)],
                      lbuf.at[pl.ds(0, FL)])

      def pstep(i, _):
        gidx = c * FL + i * 16
        eidv = ebuf[pl.ds(_mo(i * 16, 16), 16)]
        ldv = lbuf[pl.ds(_mo(i * 16, 16), 16)]
        for lane in range(16):
          @pl.when(gidx + lane < cnt)
          def _place():
            ld = jnp.max(jnp.where(iota == lane, ldv, 0))
            pos = sm[ld]
            sm[ld] = pos + 1
            plsc.store_scatter(sbufv, [zero + pos], eidv, mask=iota == lane)
        return 0

      lax.fori_loop(0, FL // 16, pstep, 0)
      return 0

    lax.fori_loop(0, nch, pchunk, 0)

    def padrow(t, _):
      lev = laste[pl.ds(_mo(t * 16, 16), 16)]
      for lane in range(16):
        r = t * 16 + lane
        cur = sm[r]
        nxt = sm[NH + r + 1]
        le = jnp.max(jnp.where(iota == lane, lev, 0))
        plsc.store_scatter(sbufv, [cur + iota], zero + le,
                           mask=iota < (nxt - cur))
        g0 = sm[NH + r] // 16

        def gcond(g):
          return g < nxt // 16

        def gfill(g):
          plsc.store_scatter(glbuf, [zero + g], zero + r, mask=iota == 0)
          return g + 1

        lax.while_loop(gcond, gfill, g0)
      return 0

    lax.fori_loop(0, NPW // 16, padrow, 0)

  for blk in range(SCAP // FL):
    pltpu.sync_copy(sbufv.at[pl.ds(blk * FL, FL)],
                    sort_hbm.at[pl.ds(_mo(w * SCAP + blk * FL), FL)])
  pltpu.sync_copy(glbuf, gl_hbm.at[pl.ds(_mo(w * NG), NG)])

  meta = jnp.where(iota == 0, cnt,
                   jnp.where(iota == 1, ok, jnp.where(iota == 2, ngr, 0)))
  dbuf[pl.ds(0, 16)] = meta
  pltpu.sync_copy(dbuf.at[pl.ds(0, 16)], cnts_hbm.at[pl.ds(_mo(w * 16, 16), 16)])


def _make_sc_gather():
  e_out = jax.ShapeDtypeStruct((N_EDGES, D), jnp.float32)

  @functools.partial(
      pl.kernel,
      out_type=e_out,
      mesh=_mesh,
      compiler_params=pltpu.CompilerParams(needs_layout_passes=False),
      scratch_types=[
          pltpu.VMEM((CG,), jnp.int32),
          pltpu.VMEM((CG,), jnp.int32),
          pltpu.VMEM((CG, D), jnp.float32),
          pltpu.VMEM((CG, D), jnp.float32),
          pltpu.SemaphoreType.DMA,
          pltpu.SemaphoreType.DMA,
      ],
  )
  def sc_gather(a_hbm, b_hbm, src_hbm, dst_hbm, g_hbm,
                dstb, srcb, ar, br, sem1, sem2):
    w = _wid()

    def chunk(c, _):
      off = w * EW + c * CG
      pltpu.sync_copy(dst_hbm.at[pl.ds(_mo(off), CG)], dstb)
      pltpu.sync_copy(src_hbm.at[pl.ds(_mo(off), CG)], srcb)
      d1 = pltpu.async_copy(a_hbm.at[dstb], ar, sem1)
      d2 = pltpu.async_copy(b_hbm.at[srcb], br, sem2)
      d1.wait()
      d2.wait()

      def row(r, _):
        for k in range(D // 16):
          sl = pl.ds(k * 16, 16)
          ar[r, sl] = ar[r, sl] + br[r, sl]
        return 0

      lax.fori_loop(0, CG, row, 0)
      pltpu.sync_copy(ar, g_hbm.at[pl.ds(_mo(off), CG)])
      return 0

    lax.fori_loop(0, EW // CG, chunk, 0)

  return sc_gather


_sc_gather = _make_sc_gather()


def _make_sc_scatter_max():
  out = jax.ShapeDtypeStruct((NP, D), jnp.float32)

  @functools.partial(
      pl.kernel,
      out_type=out,
      mesh=_mesh,
      compiler_params=pltpu.CompilerParams(needs_layout_passes=False),
      scratch_types=[
          pltpu.VMEM((FL,), jnp.int32),
          pltpu.VMEM((FL,), jnp.int32),
          pltpu.VMEM((128, D), jnp.float32),
          pltpu.VMEM(((NPW + 1) * D,), jnp.float32),
          pltpu.VMEM((CG, D), jnp.float32),
          pltpu.VMEM((16,), jnp.int32),
          pltpu.SMEM((FL // 16,), jnp.int32),
          pltpu.SemaphoreType.DMA,
      ],
  )
  def sc_scatter(m_hbm, eids_hbm, lds_hbm, cnts_hbm, sort_hbm, gl_hbm,
                 s_hbm, hn_hbm, eidb, ldb, mbuf, acc, sbuf, cntb, sm, sem):
    w = _wid()
    iota = lax.iota(jnp.int32, 16)

    for q in range(NPW // CG):
      pltpu.sync_copy(s_hbm.at[pl.ds(_mo(w * NPW + q * CG), CG)], sbuf)

      def irow(r, _):
        for k in range(D // 16):
          acc[pl.ds(_mo((q * CG + r) * D + k * 16, 16), 16)] = jnp.maximum(
              sbuf[r, pl.ds(k * 16, 16)], 0.0)
        return 0

      lax.fori_loop(0, CG, irow, 0)

    pltpu.sync_copy(cnts_hbm.at[pl.ds(_mo(w * 16, 16), 16)], cntb)
    mv = cntb[...]
    cnt = jnp.max(jnp.where(iota == 0, mv, 0))
    ok = jnp.max(jnp.where(iota == 1, mv, 0))
    ngr = jnp.max(jnp.where(iota == 2, mv, 0))

    @pl.when(ok == 1)
    def _fast():
      nch = (ngr * 16 + FL - 1) // FL

      def chunk(c, _):
        pltpu.sync_copy(sort_hbm.at[pl.ds(_mo(w * SCAP + c * FL), FL)], eidb)
        glv = gl_hbm.at[pl.ds(_mo(w * NG + c * (FL // 16)), FL // 16)]
        pltpu.sync_copy(glv, ldb.at[pl.ds(0, FL // 16)])

        def gl2sm(t, _):
          gv = ldb[pl.ds(_mo(t * 16, 16), 16)]
          for lane in range(16):
            sm[t * 16 + lane] = jnp.max(jnp.where(iota == lane, gv, 0))
          return 0

        lax.fori_loop(0, FL // 256, gl2sm, 0)

        def sub(s, _):
          pltpu.async_copy(
              m_hbm.at[eidb.at[pl.ds(_mo(s * 128), 128)]], mbuf, sem).wait()
          for gg in range(8):
            r = sm[s * 8 + gg]
            rb = r * D
            for k in range(D // 16):
              sl = pl.ds(k * 16, 16)
              vals = [mbuf[gg * 16 + u, sl] for u in range(16)]
              while len(vals) > 1:
                vals = [jnp.maximum(vals[2 * i], vals[2 * i + 1])
                        for i in range(len(vals) // 2)]
              asl = pl.ds(_mo(rb + k * 16, 16), 16)
              acc[asl] = jnp.maximum(acc[asl], vals[0])
          return 0

        lax.fori_loop(0, FL // 128, sub, 0)
        return 0

      lax.fori_loop(0, nch, chunk, 0)

    @pl.when(ok == 0)
    def _slow():
      nch = (cnt + FL - 1) // FL

      def chunk(c, _):
        pltpu.sync_copy(eids_hbm.at[pl.ds(_mo(w * CAPW + c * FL), FL)], eidb)
        pltpu.sync_copy(lds_hbm.at[pl.ds(_mo(w * CAPW + c * FL), FL)], ldb)

        def sub(s, _):
          pltpu.async_copy(
              m_hbm.at[eidb.at[pl.ds(_mo(s * 128), 128)]], mbuf, sem).wait()

          def edge(j, _):
            ldv = ldb[pl.ds(_mo(s * 128 + jnp.bitwise_and(j, ~15), 16), 16)]
            lane = jnp.bitwise_and(j, 15)
            ld = jnp.max(jnp.where(iota == lane, ldv, 0))
            rb = ld * D
            for k in range(D // 16):
              sl = pl.ds(_mo(rb + k * 16, 16), 16)
              acc[sl] = jnp.maximum(acc[sl], mbuf[j, pl.ds(k * 16, 16)])
            return 0

          lax.fori_loop(0, 128, edge, 0)
          return 0

        lax.fori_loop(0, FL // 128, sub, 0)
        return 0

      lax.fori_loop(0, nch, chunk, 0)

    for q in range(NPW // CG):
      def orow(r, _):
        for k in range(D // 16):
          sbuf[r, pl.ds(k * 16, 16)] = acc[pl.ds(_mo((q * CG + r) * D + k * 16, 16), 16)]
        return 0

      lax.fori_loop(0, CG, orow, 0)
      pltpu.sync_copy(sbuf, hn_hbm.at[pl.ds(_mo(w * NPW + q * CG), CG)])

  return sc_scatter


_sc_scatter_max = _make_sc_scatter_max()


@jax.jit
def kernel(x, edge_index, edge_attr, W_enc, b_enc,
           W1_0, b1_0, W2_0, b2_0,
           W1_1, b1_1, W2_1, b2_1,
           W1_2, b1_2, W2_2, b2_2,
           W_dec, b_dec):
  src = edge_index[0]
  dst = edge_index[1]
  eap = jnp.pad(edge_attr, ((0, 0), (0, 5)))

  eids, lds, cnts, sort_e, gl = _sc_bin(dst)

  xp = jnp.pad(x, ((0, NP - N_NODES), (0, 0)))
  h = _linear(xp, W_enc.T, b_enc[None, :], act=True)

  for (W1, b1, W2, b2) in ((W1_0, b1_0, W2_0, b2_0),
                           (W1_1, b1_1, W2_1, b2_1),
                           (W1_2, b1_2, W2_2, b2_2)):
    w1it = W1[:, :H].T
    w1jt = W1[:, H:2 * H].T
    w1et = jnp.pad(W1[:, 2 * H:], ((0, 0), (0, 5))).T
    a, b, s = _node_kernel(h, w1it, w1jt, b1[None, :], W2.T, b2[None, :])
    g = _sc_gather(a, b, src, dst)
    m = _edge_kernel(g, eap, w1et, b1[None, :], W2.T, b2[None, :])
    h = _sc_scatter_max(m, eids, lds, cnts, sort_e, gl, s)

  out = _linear(h, W_dec.T, b_dec[None, :], act=False)
  return out[:N_NODES]


# speedup vs baseline: 1.0254x; 1.0254x over previous
import functools
import jax
import jax.numpy as jnp
from jax import lax
from jax.experimental import pallas as pl
from jax.experimental.pallas import tpu as pltpu
from jax.experimental.pallas import tpu_sc as plsc

N_NODES = 10000
N_EDGES = 320000
D = 128
H = 128

NODE_BLK = 512
EDGE_BLK = 512
NP = 10240

NC = 2
NS = 16
NW = NC * NS
NPW = NP // NW
EW = N_EDGES // NW

FL = 2048
CBUF = 8192
CAPW = N_EDGES + FL
CH = 3200
CG = 80
EID_BITS = 19
EID_MASK = (1 << EID_BITS) - 1

SCAP = 24576
NG = SCAP // 16
NH = 336

_INTERPRET = False

_mesh = plsc.VectorSubcoreMesh(
    core_axis_name="c", subcore_axis_name="s", num_cores=NC, num_subcores=NS)


def _mo(v, n=8):
  return pl.multiple_of(v, n)


def _wid():
  return lax.axis_index("s") * NC + lax.axis_index("c")


def _linear_body(x_ref, wt_ref, b_ref, o_ref, *, act):
  y = jnp.dot(x_ref[...], wt_ref[...], preferred_element_type=jnp.float32)
  y = y + b_ref[...]
  if act:
    y = jnp.maximum(y, 0.0)
  o_ref[...] = y


def _linear(x, wt, b, act):
  n = x.shape[0]
  grid = n // NODE_BLK
  return pl.pallas_call(
      functools.partial(_linear_body, act=act),
      grid=(grid,),
      in_specs=[
          pl.BlockSpec((NODE_BLK, D), lambda i: (i, 0)),
          pl.BlockSpec((D, D), lambda i: (0, 0)),
          pl.BlockSpec((1, D), lambda i: (0, 0)),
      ],
      out_specs=pl.BlockSpec((NODE_BLK, D), lambda i: (i, 0)),
      out_shape=jax.ShapeDtypeStruct((n, D), jnp.float32),
      interpret=_INTERPRET,
  )(x, wt, b)


def _node_body(h_ref, w1it_ref, w1jt_ref, b1_ref, w2t_ref, b2_ref,
               a_ref, b_ref, s_ref):
  h = h_ref[...]
  a = jnp.dot(h, w1it_ref[...], preferred_element_type=jnp.float32)
  b = jnp.dot(h, w1jt_ref[...], preferred_element_type=jnp.float32)
  a_ref[...] = a
  b_ref[...] = b
  t = jnp.maximum(a + b + b1_ref[...], 0.0)
  s_ref[...] = jnp.dot(t, w2t_ref[...],
                       preferred_element_type=jnp.float32) + b2_ref[...]


def _node_kernel(h, w1it, w1jt, b1, w2t, b2):
  n = h.shape[0]
  grid = n // NODE_BLK
  out = jax.ShapeDtypeStruct((n, D), jnp.float32)
  return pl.pallas_call(
      _node_body,
      grid=(grid,),
      in_specs=[
          pl.BlockSpec((NODE_BLK, D), lambda i: (i, 0)),
          pl.BlockSpec((D, D), lambda i: (0, 0)),
          pl.BlockSpec((D, D), lambda i: (0, 0)),
          pl.BlockSpec((1, D), lambda i: (0, 0)),
          pl.BlockSpec((D, D), lambda i: (0, 0)),
          pl.BlockSpec((1, D), lambda i: (0, 0)),
      ],
      out_specs=[pl.BlockSpec((NODE_BLK, D), lambda i: (i, 0))] * 3,
      out_shape=[out, out, out],
      interpret=_INTERPRET,
  )(h, w1it, w1jt, b1, w2t, b2)


def _edge_body(g_ref, ea_ref, w1et_ref, b1_ref, w2t_ref, b2_ref, m_ref):
  t = g_ref[...] + jnp.dot(ea_ref[...], w1et_ref[...],
                           preferred_element_type=jnp.float32) + b1_ref[...]
  t = jnp.maximum(t, 0.0)
  m_ref[...] = jnp.dot(t, w2t_ref[...],
                       preferred_element_type=jnp.float32) + b2_ref[...]


def _edge_kernel(g, eap, w1et, b1, w2t, b2):
  e = g.shape[0]
  grid = e // EDGE_BLK
  return pl.pallas_call(
      _edge_body,
      grid=(grid,),
      in_specs=[
          pl.BlockSpec((EDGE_BLK, D), lambda i: (i, 0)),
          pl.BlockSpec((EDGE_BLK, 8), lambda i: (i, 0)),
          pl.BlockSpec((8, D), lambda i: (0, 0)),
          pl.BlockSpec((1, D), lambda i: (0, 0)),
          pl.BlockSpec((D, D), lambda i: (0, 0)),
          pl.BlockSpec((1, D), lambda i: (0, 0)),
      ],
      out_specs=pl.BlockSpec((EDGE_BLK, D), lambda i: (i, 0)),
      out_shape=jax.ShapeDtypeStruct((e, D), jnp.float32),
      interpret=_INTERPRET,
  )(g, eap, w1et, b1, w2t, b2)


@functools.partial(
    pl.kernel,
    out_type=[jax.ShapeDtypeStruct((NW * CAPW,), jnp.int32),
              jax.ShapeDtypeStruct((NW * CAPW,), jnp.int32),
              jax.ShapeDtypeStruct((NW * 16,), jnp.int32),
              jax.ShapeDtypeStruct((NW * SCAP,), jnp.int32),
              jax.ShapeDtypeStruct((NW * NG,), jnp.int32)],
    mesh=_mesh,
    compiler_params=pltpu.CompilerParams(needs_layout_passes=False),
    scratch_types=[
        pltpu.VMEM((CH,), jnp.int32),
        pltpu.VMEM((CBUF,), jnp.int32),
        pltpu.VMEM((CBUF,), jnp.int32),
        pltpu.VMEM((NH,), jnp.int32),
        pltpu.VMEM((NH,), jnp.int32),
        pltpu.VMEM((NH,), jnp.int32),
        pltpu.VMEM((SCAP,), jnp.int32),
        pltpu.VMEM((NG,), jnp.int32),
        pltpu.SMEM((2 * NH,), jnp.int32),
    ],
)
def _sc_bin(dst_hbm, eids_hbm, lds_hbm, cnts_hbm, sort_hbm, gl_hbm,
            dbuf, ebuf, lbuf, histv, laste, bbuf, sbufv, glbuf, sm):
  w = _wid()
  base = w * NPW
  lim = base + NPW
  iota = lax.iota(jnp.int32, 16)
  sent_ld = jnp.full((16,), NPW, jnp.int32)
  zero = jnp.full((16,), 0, jnp.int32)
  ones = jnp.full((16,), 1, jnp.int32)

  def init(i, _):
    ebuf[pl.ds(_mo(i * 16, 16), 16)] = zero
    lbuf[pl.ds(_mo(i * 16, 16), 16)] = sent_ld
    return 0
  lax.fori_loop(0, CBUF // 16, init, 0)

  def init2(i, _):
    sbufv[pl.ds(_mo(i * 16, 16), 16)] = zero
    return 0
  lax.fori_loop(0, SCAP // 16, init2, 0)

  def init3(i, _):
    glbuf[pl.ds(_mo(i * 16, 16), 16)] = sent_ld
    return 0
  lax.fori_loop(0, NG // 16, init3, 0)

  for t in range(NH // 16):
    histv[pl.ds(t * 16, 16)] = zero
    laste[pl.ds(t * 16, 16)] = zero

  def chunk(c, carry):
    p, fl, cnt = carry
    pltpu.sync_copy(dst_hbm.at[pl.ds(_mo(c * CH), CH)], dbuf)

    def step(i, carry2):
      p, cnt = carry2
      v = dbuf[pl.ds(_mo(i * 16, 16), 16)]
      eidv = c * CH + i * 16 + iota
      msk = jnp.logical_and(v >= base, v < lim)
      ldv = v - base
      plsc.store_compressed(ebuf.at[pl.ds(p, 16)], eidv, mask=msk)
      plsc.store_compressed(lbuf.at[pl.ds(p, 16)], ldv, mask=msk)
      plsc.addupdate_scatter(histv, [jnp.where(msk, ldv, NPW)], ones, mask=msk)
      plsc.store_scatter(laste, [jnp.where(msk, ldv, NPW)], eidv, mask=msk)
      pc = jnp.max(plsc.all_reduce_population_count(msk))
      return p + pc, cnt + pc

    p, cnt = lax.fori_loop(0, CH // 16, step, (p, cnt))

    def flush_cond(fc):
      return fc[0] >= FL

    def flush(fc):
      p, fl = fc
      pltpu.sync_copy(ebuf.at[pl.ds(0, FL)],
                      eids_hbm.at[pl.ds(_mo(w * CAPW + fl), FL)])
      pltpu.sync_copy(lbuf.at[pl.ds(0, FL)],
                      lds_hbm.at[pl.ds(_mo(w * CAPW + fl), FL)])

      def shift(i, _):
        ebuf[pl.ds(_mo(i * 16, 16), 16)] = ebuf[pl.ds(_mo(i * 16 + FL, 16), 16)]
        lbuf[pl.ds(_mo(i * 16, 16), 16)] = lbuf[pl.ds(_mo(i * 16 + FL, 16), 16)]
        return 0

      lax.fori_loop(0, (CBUF - FL) // 16, shift, 0)
      return p - FL, fl + FL

    p, fl = lax.while_loop(flush_cond, flush, (p, fl))
    return p, fl, cnt

  p, fl, cnt = lax.fori_loop(0, N_EDGES // CH, chunk, (0, 0, 0))

  def drain_cond(fc):
    return fc[0] > 0

  def drain(fc):
    p, fl = fc
    pltpu.sync_copy(ebuf.at[pl.ds(0, FL)],
                    eids_hbm.at[pl.ds(_mo(w * CAPW + fl), FL)])
    pltpu.sync_copy(lbuf.at[pl.ds(0, FL)],
                    lds_hbm.at[pl.ds(_mo(w * CAPW + fl), FL)])
    return p - FL, fl + FL

  p, fl = lax.while_loop(drain_cond, drain, (p, fl))

  def bases(t, carry):
    h = histv[pl.ds(_mo(t * 16, 16), 16)]
    hp = jnp.bitwise_and(h + 15, ~15)
    cs = plsc.cumsum(hp)
    bbuf[pl.ds(_mo(t * 16, 16), 16)] = cs - hp + carry
    return carry + jnp.max(cs)

  total = lax.fori_loop(0, NH // 16, bases, 0)
  ok = jnp.int32(total <= SCAP)
  ngr = total // 16

  def extract(t, _):
    bv = bbuf[pl.ds(_mo(t * 16, 16), 16)]
    for lane in range(16):
      s = jnp.max(jnp.where(iota == lane, bv, 0))
      sm[t * 16 + lane] = s
      sm[NH + t * 16 + lane] = s
    return 0

  lax.fori_loop(0, NH // 16, extract, 0)

  @pl.when(ok == 1)
  def _sort():
    nch = (cnt + FL - 1) // FL

    def pchunk(c, _):
      pltpu.sync_copy(eids_hbm.at[pl.ds(_mo(w * CAPW + c * FL), FL)],
                      ebuf.at[pl.ds(0, FL)])
      pltpu.sync_copy(lds_hbm.at[pl.ds(_mo(w * CAPW + c * FL), FL)],
                      lbuf.at[pl.ds(0, FL)])

      def pstep(i, _):
        gidx = c * FL + i * 16
        eidv = ebuf[pl.ds(_mo(i * 16, 16), 16)]
        ldv = lbuf[pl.ds(_mo(i * 16, 16), 16)]
        for lane in range(16):
          @pl.when(gidx + lane < cnt)
          def _place():
            ld = jnp.max(jnp.where(iota == lane, ldv, 0))
            pos = sm[ld]
            sm[ld] = pos + 1
            plsc.store_scatter(sbufv, [zero + pos], eidv, mask=iota == lane)
        return 0

      lax.fori_loop(0, FL // 16, pstep, 0)
      return 0

    lax.fori_loop(0, nch, pchunk, 0)

    def padrow(t, _):
      lev = laste[pl.ds(_mo(t * 16, 16), 16)]
      for lane in range(16):
        r = t * 16 + lane
        cur = sm[r]
        nxt = sm[NH + r + 1]
        le = jnp.max(jnp.where(iota == lane, lev, 0))
        plsc.store_scatter(sbufv, [cur + iota], zero + le,
                           mask=iota < (nxt - cur))
        g0 = sm[NH + r] // 16

        def gcond(g):
          return g < nxt // 16

        def gfill(g):
          plsc.store_scatter(glbuf, [zero + g], zero + r, mask=iota == 0)
          return g + 1

        lax.while_loop(gcond, gfill, g0)
      return 0

    lax.fori_loop(0, NPW // 16, padrow, 0)

  for blk in range(SCAP // FL):
    pltpu.sync_copy(sbufv.at[pl.ds(blk * FL, FL)],
                    sort_hbm.at[pl.ds(_mo(w * SCAP + blk * FL), FL)])
  pltpu.sync_copy(glbuf, gl_hbm.at[pl.ds(_mo(w * NG), NG)])

  meta = jnp.where(iota == 0, cnt,
                   jnp.where(iota == 1, ok, jnp.where(iota == 2, ngr, 0)))
  dbuf[pl.ds(0, 16)] = meta
  pltpu.sync_copy(dbuf.at[pl.ds(0, 16)], cnts_hbm.at[pl.ds(_mo(w * 16, 16), 16)])


def _make_sc_gather():
  e_out = jax.ShapeDtypeStruct((N_EDGES, D), jnp.float32)

  @functools.partial(
      pl.kernel,
      out_type=e_out,
      mesh=_mesh,
      compiler_params=pltpu.CompilerParams(needs_layout_passes=False),
      scratch_types=[
          pltpu.VMEM((CG,), jnp.int32),
          pltpu.VMEM((CG,), jnp.int32),
          pltpu.VMEM((CG, D), jnp.float32),
          pltpu.VMEM((CG, D), jnp.float32),
          pltpu.SemaphoreType.DMA,
          pltpu.SemaphoreType.DMA,
      ],
  )
  def sc_gather(a_hbm, b_hbm, src_hbm, dst_hbm, g_hbm,
                dstb, srcb, ar, br, sem1, sem2):
    w = _wid()

    def chunk(c, _):
      off = w * EW + c * CG
      pltpu.sync_copy(dst_hbm.at[pl.ds(_mo(off), CG)], dstb)
      pltpu.sync_copy(src_hbm.at[pl.ds(_mo(off), CG)], srcb)
      d1 = pltpu.async_copy(a_hbm.at[dstb], ar, sem1)
      d2 = pltpu.async_copy(b_hbm.at[srcb], br, sem2)
      d1.wait()
      d2.wait()

      def row(r, _):
        for k in range(D // 16):
          sl = pl.ds(k * 16, 16)
          ar[r, sl] = ar[r, sl] + br[r, sl]
        return 0

      lax.fori_loop(0, CG, row, 0)
      pltpu.sync_copy(ar, g_hbm.at[pl.ds(_mo(off), CG)])
      return 0

    lax.fori_loop(0, EW // CG, chunk, 0)

  return sc_gather


_sc_gather = _make_sc_gather()


def _make_sc_scatter_max():
  out = jax.ShapeDtypeStruct((NP, D), jnp.float32)

  @functools.partial(
      pl.kernel,
      out_type=out,
      mesh=_mesh,
      compiler_params=pltpu.CompilerParams(needs_layout_passes=False),
      scratch_types=[
          pltpu.VMEM((FL,), jnp.int32),
          pltpu.VMEM((FL,), jnp.int32),
          pltpu.VMEM((128, D), jnp.float32),
          pltpu.VMEM((128, D), jnp.float32),
          pltpu.VMEM((128, D), jnp.float32),
          pltpu.VMEM(((NPW + 1) * D,), jnp.float32),
          pltpu.VMEM((CG, D), jnp.float32),
          pltpu.VMEM((16,), jnp.int32),
          pltpu.SMEM((FL // 16,), jnp.int32),
          pltpu.SemaphoreType.DMA,
          pltpu.SemaphoreType.DMA,
          pltpu.SemaphoreType.DMA,
      ],
  )
  def sc_scatter(m_hbm, eids_hbm, lds_hbm, cnts_hbm, sort_hbm, gl_hbm,
                 s_hbm, hn_hbm, eidb, ldb, mb0, mb1, mb2, acc, sbuf, cntb, sm,
                 sem0, sem1, sem2):
    mbufs = (mb0, mb1, mb2)
    sems = (sem0, sem1, sem2)
    mbuf = mb0
    sem = sem0
    w = _wid()
    iota = lax.iota(jnp.int32, 16)

    for q in range(NPW // CG):
      pltpu.sync_copy(s_hbm.at[pl.ds(_mo(w * NPW + q * CG), CG)], sbuf)

      def irow(r, _):
        for k in range(D // 16):
          acc[pl.ds(_mo((q * CG + r) * D + k * 16, 16), 16)] = jnp.maximum(
              sbuf[r, pl.ds(k * 16, 16)], 0.0)
        return 0

      lax.fori_loop(0, CG, irow, 0)

    pltpu.sync_copy(cnts_hbm.at[pl.ds(_mo(w * 16, 16), 16)], cntb)
    mv = cntb[...]
    cnt = jnp.max(jnp.where(iota == 0, mv, 0))
    ok = jnp.max(jnp.where(iota == 1, mv, 0))
    ngr = jnp.max(jnp.where(iota == 2, mv, 0))

    @pl.when(ok == 1)
    def _fast():
      nch = (ngr * 16 + FL - 1) // FL

      def chunk(c, _):
        pltpu.sync_copy(sort_hbm.at[pl.ds(_mo(w * SCAP + c * FL), FL)], eidb)
        glv = gl_hbm.at[pl.ds(_mo(w * NG + c * (FL // 16)), FL // 16)]
        pltpu.sync_copy(glv, ldb.at[pl.ds(0, FL // 16)])

        def gl2sm(t, _):
          gv = ldb[pl.ds(_mo(t * 16, 16), 16)]
          for lane in range(16):
            sm[t * 16 + lane] = jnp.max(jnp.where(iota == lane, gv, 0))
          return 0

        lax.fori_loop(0, FL // 256, gl2sm, 0)

        for b in range(3):
          pltpu.async_copy(
              m_hbm.at[eidb.at[pl.ds(_mo(b * 128), 128)]], mbufs[b], sems[b])

        def sub3(s0, _):
          for par in range(3):
            s = s0 * 3 + par
            mb = mbufs[par]
            pltpu.make_async_copy(
                m_hbm.at[eidb.at[pl.ds(_mo(s * 128), 128)]], mb,
                sems[par]).wait()
            for gg in range(8):
              r = sm[s * 8 + gg]
              rb = r * D
              for k in range(D // 16):
                sl = pl.ds(k * 16, 16)
                vals = [mb[gg * 16 + u, sl] for u in range(16)]
                while len(vals) > 1:
                  vals = [jnp.maximum(vals[2 * i], vals[2 * i + 1])
                          for i in range(len(vals) // 2)]
                asl = pl.ds(_mo(rb + k * 16, 16), 16)
                acc[asl] = jnp.maximum(acc[asl], vals[0])

            @pl.when(s + 3 < FL // 128)
            def _prefetch():
              pltpu.async_copy(
                  m_hbm.at[eidb.at[pl.ds(_mo((s + 3) * 128), 128)]], mb,
                  sems[par])
          return 0

        lax.fori_loop(0, FL // (128 * 3), sub3, 0)

        s = 15
        mb = mbufs[s % 3]
        pltpu.make_async_copy(
            m_hbm.at[eidb.at[pl.ds(_mo(s * 128), 128)]], mb, sems[s % 3]).wait()
        for gg in range(8):
          r = sm[s * 8 + gg]
          rb = r * D
          for k in range(D // 16):
            sl = pl.ds(k * 16, 16)
            vals = [mb[gg * 16 + u, sl] for u in range(16)]
            while len(vals) > 1:
              vals = [jnp.maximum(vals[2 * i], vals[2 * i + 1])
                      for i in range(len(vals) // 2)]
            asl = pl.ds(_mo(rb + k * 16, 16), 16)
            acc[asl] = jnp.maximum(acc[asl], vals[0])
        return 0

      lax.fori_loop(0, nch, chunk, 0)

    @pl.when(ok == 0)
    def _slow():
      nch = (cnt + FL - 1) // FL

      def chunk(c, _):
        pltpu.sync_copy(eids_hbm.at[pl.ds(_mo(w * CAPW + c * FL), FL)], eidb)
        pltpu.sync_copy(lds_hbm.at[pl.ds(_mo(w * CAPW + c * FL), FL)], ldb)

        def sub(s, _):
          pltpu.async_copy(
              m_hbm.at[eidb.at[pl.ds(_mo(s * 128), 128)]], mbuf, sem).wait()

          def edge(j, _):
            ldv = ldb[pl.ds(_mo(s * 128 + jnp.bitwise_and(j, ~15), 16), 16)]
            lane = jnp.bitwise_and(j, 15)
            ld = jnp.max(jnp.where(iota == lane, ldv, 0))
            rb = ld * D
            for k in range(D // 16):
              sl = pl.ds(_mo(rb + k * 16, 16), 16)
              acc[sl] = jnp.maximum(acc[sl], mbuf[j, pl.ds(k * 16, 16)])
            return 0

          lax.fori_loop(0, 128, edge, 0)
          return 0

        lax.fori_loop(0, FL // 128, sub, 0)
        return 0

      lax.fori_loop(0, nch, chunk, 0)

    for q in range(NPW // CG):
      def orow(r, _):
        for k in range(D // 16):
          sbuf[r, pl.ds(k * 16, 16)] = acc[pl.ds(_mo((q * CG + r) * D + k * 16, 16), 16)]
        return 0

      lax.fori_loop(0, CG, orow, 0)
      pltpu.sync_copy(sbuf, hn_hbm.at[pl.ds(_mo(w * NPW + q * CG), CG)])

  return sc_scatter


_sc_scatter_max = _make_sc_scatter_max()


@jax.jit
def kernel(x, edge_index, edge_attr, W_enc, b_enc,
           W1_0, b1_0, W2_0, b2_0,
           W1_1, b1_1, W2_1, b2_1,
           W1_2, b1_2, W2_2, b2_2,
           W_dec, b_dec):
  src = edge_index[0]
  dst = edge_index[1]
  eap = jnp.pad(edge_attr, ((0, 0), (0, 5)))

  eids, lds, cnts, sort_e, gl = _sc_bin(dst)

  xp = jnp.pad(x, ((0, NP - N_NODES), (0, 0)))
  h = _linear(xp, W_enc.T, b_enc[None, :], act=True)

  for (W1, b1, W2, b2) in ((W1_0, b1_0, W2_0, b2_0),
                           (W1_1, b1_1, W2_1, b2_1),
                           (W1_2, b1_2, W2_2, b2_2)):
    w1it = W1[:, :H].T
    w1jt = W1[:, H:2 * H].T
    w1et = jnp.pad(W1[:, 2 * H:], ((0, 0), (0, 5))).T
    a, b, s = _node_kernel(h, w1it, w1jt, b1[None, :], W2.T, b2[None, :])
    g = _sc_gather(a, b, src, dst)
    m = _edge_kernel(g, eap, w1et, b1[None, :], W2.T, b2[None, :])
    h = _sc_scatter_max(m, eids, lds, cnts, sort_e, gl, s)

  out = _linear(h, W_dec.T, b_dec[None, :], act=False)
  return out[:N_NODES]


# speedup vs baseline: 1.2799x; 1.2482x over previous
import functools
import jax
import jax.numpy as jnp
from jax import lax
from jax.experimental import pallas as pl
from jax.experimental.pallas import tpu as pltpu
from jax.experimental.pallas import tpu_sc as plsc

N_NODES = 10000
N_EDGES = 320000
D = 128
H = 128

NODE_BLK = 512
EDGE_BLK = 512
NP = 10240

NC = 2
NS = 16
NW = NC * NS
NPW = NP // NW
EW = N_EDGES // NW

FL = 2048
CBUF = 8192
CAPW = N_EDGES + FL
CH = 3200
CG = 80
EID_BITS = 19
EID_MASK = (1 << EID_BITS) - 1

SCAP = 24576
NG = SCAP // 16
NH = 336

_INTERPRET = False

_mesh = plsc.VectorSubcoreMesh(
    core_axis_name="c", subcore_axis_name="s", num_cores=NC, num_subcores=NS)


def _mo(v, n=8):
  return pl.multiple_of(v, n)


def _wid():
  return lax.axis_index("s") * NC + lax.axis_index("c")


def _linear_body(x_ref, wt_ref, b_ref, o_ref, *, act):
  y = jnp.dot(x_ref[...], wt_ref[...], preferred_element_type=jnp.float32)
  y = y + b_ref[...]
  if act:
    y = jnp.maximum(y, 0.0)
  o_ref[...] = y


def _linear(x, wt, b, act):
  n = x.shape[0]
  grid = n // NODE_BLK
  return pl.pallas_call(
      functools.partial(_linear_body, act=act),
      grid=(grid,),
      in_specs=[
          pl.BlockSpec((NODE_BLK, D), lambda i: (i, 0)),
          pl.BlockSpec((D, D), lambda i: (0, 0)),
          pl.BlockSpec((1, D), lambda i: (0, 0)),
      ],
      out_specs=pl.BlockSpec((NODE_BLK, D), lambda i: (i, 0)),
      out_shape=jax.ShapeDtypeStruct((n, D), jnp.float32),
      interpret=_INTERPRET,
  )(x, wt, b)


def _node_body(h_ref, w1it_ref, w1jt_ref, b1_ref, w2t_ref, b2_ref,
               a_ref, b_ref, s_ref):
  h = h_ref[...]
  a = jnp.dot(h, w1it_ref[...], preferred_element_type=jnp.float32)
  b = jnp.dot(h, w1jt_ref[...], preferred_element_type=jnp.float32)
  a_ref[...] = a
  b_ref[...] = b
  t = jnp.maximum(a + b + b1_ref[...], 0.0)
  s_ref[...] = jnp.dot(t, w2t_ref[...],
                       preferred_element_type=jnp.float32) + b2_ref[...]


def _node_kernel(h, w1it, w1jt, b1, w2t, b2):
  n = h.shape[0]
  grid = n // NODE_BLK
  out = jax.ShapeDtypeStruct((n, D), jnp.float32)
  return pl.pallas_call(
      _node_body,
      grid=(grid,),
      in_specs=[
          pl.BlockSpec((NODE_BLK, D), lambda i: (i, 0)),
          pl.BlockSpec((D, D), lambda i: (0, 0)),
          pl.BlockSpec((D, D), lambda i: (0, 0)),
          pl.BlockSpec((1, D), lambda i: (0, 0)),
          pl.BlockSpec((D, D), lambda i: (0, 0)),
          pl.BlockSpec((1, D), lambda i: (0, 0)),
      ],
      out_specs=[pl.BlockSpec((NODE_BLK, D), lambda i: (i, 0))] * 3,
      out_shape=[out, out, out],
      interpret=_INTERPRET,
  )(h, w1it, w1jt, b1, w2t, b2)


def _edge_body(g_ref, ea_ref, w1et_ref, b1_ref, w2t_ref, b2_ref, m_ref):
  t = g_ref[...] + jnp.dot(ea_ref[...], w1et_ref[...],
                           preferred_element_type=jnp.float32) + b1_ref[...]
  t = jnp.maximum(t, 0.0)
  m_ref[...] = jnp.dot(t, w2t_ref[...],
                       preferred_element_type=jnp.float32) + b2_ref[...]


def _edge_kernel(g, eap, w1et, b1, w2t, b2):
  e = g.shape[0]
  grid = e // EDGE_BLK
  return pl.pallas_call(
      _edge_body,
      grid=(grid,),
      in_specs=[
          pl.BlockSpec((EDGE_BLK, D), lambda i: (i, 0)),
          pl.BlockSpec((EDGE_BLK, 8), lambda i: (i, 0)),
          pl.BlockSpec((8, D), lambda i: (0, 0)),
          pl.BlockSpec((1, D), lambda i: (0, 0)),
          pl.BlockSpec((D, D), lambda i: (0, 0)),
          pl.BlockSpec((1, D), lambda i: (0, 0)),
      ],
      out_specs=pl.BlockSpec((EDGE_BLK, D), lambda i: (i, 0)),
      out_shape=jax.ShapeDtypeStruct((e, D), jnp.float32),
      interpret=_INTERPRET,
  )(g, eap, w1et, b1, w2t, b2)


@functools.partial(
    pl.kernel,
    out_type=[jax.ShapeDtypeStruct((NW * CAPW,), jnp.int32),
              jax.ShapeDtypeStruct((NW * CAPW,), jnp.int32),
              jax.ShapeDtypeStruct((NW * 16,), jnp.int32),
              jax.ShapeDtypeStruct((NW * SCAP,), jnp.int32),
              jax.ShapeDtypeStruct((NW * NG,), jnp.int32)],
    mesh=_mesh,
    compiler_params=pltpu.CompilerParams(needs_layout_passes=False),
    scratch_types=[
        pltpu.VMEM((CH,), jnp.int32),
        pltpu.VMEM((CBUF,), jnp.int32),
        pltpu.VMEM((CBUF,), jnp.int32),
        pltpu.VMEM((NH,), jnp.int32),
        pltpu.VMEM((NH,), jnp.int32),
        pltpu.VMEM((NH,), jnp.int32),
        pltpu.VMEM((SCAP,), jnp.int32),
        pltpu.VMEM((NG,), jnp.int32),
        pltpu.SMEM((2 * NH,), jnp.int32),
    ],
)
def _sc_bin(dst_hbm, eids_hbm, lds_hbm, cnts_hbm, sort_hbm, gl_hbm,
            dbuf, ebuf, lbuf, histv, laste, bbuf, sbufv, glbuf, sm):
  w = _wid()
  base = w * NPW
  lim = base + NPW
  iota = lax.iota(jnp.int32, 16)
  sent_ld = jnp.full((16,), NPW, jnp.int32)
  zero = jnp.full((16,), 0, jnp.int32)
  ones = jnp.full((16,), 1, jnp.int32)

  def init(i, _):
    ebuf[pl.ds(_mo(i * 16, 16), 16)] = zero
    lbuf[pl.ds(_mo(i * 16, 16), 16)] = sent_ld
    return 0
  lax.fori_loop(0, CBUF // 16, init, 0)

  def init2(i, _):
    sbufv[pl.ds(_mo(i * 16, 16), 16)] = zero
    return 0
  lax.fori_loop(0, SCAP // 16, init2, 0)

  def init3(i, _):
    glbuf[pl.ds(_mo(i * 16, 16), 16)] = sent_ld
    return 0
  lax.fori_loop(0, NG // 16, init3, 0)

  for t in range(NH // 16):
    histv[pl.ds(t * 16, 16)] = zero
    laste[pl.ds(t * 16, 16)] = zero

  def chunk(c, carry):
    p, fl, cnt = carry
    pltpu.sync_copy(dst_hbm.at[pl.ds(_mo(c * CH), CH)], dbuf)

    def step(i, carry2):
      p, cnt = carry2
      v = dbuf[pl.ds(_mo(i * 16, 16), 16)]
      eidv = c * CH + i * 16 + iota
      msk = jnp.logical_and(v >= base, v < lim)
      ldv = v - base
      plsc.store_compressed(ebuf.at[pl.ds(p, 16)], eidv, mask=msk)
      plsc.store_compressed(lbuf.at[pl.ds(p, 16)], ldv, mask=msk)
      plsc.addupdate_scatter(histv, [jnp.where(msk, ldv, NPW)], ones, mask=msk)
      plsc.store_scatter(laste, [jnp.where(msk, ldv, NPW)], eidv, mask=msk)
      pc = jnp.max(plsc.all_reduce_population_count(msk))
      return p + pc, cnt + pc

    p, cnt = lax.fori_loop(0, CH // 16, step, (p, cnt))

    def flush_cond(fc):
      return fc[0] >= FL

    def flush(fc):
      p, fl = fc
      pltpu.sync_copy(ebuf.at[pl.ds(0, FL)],
                      eids_hbm.at[pl.ds(_mo(w * CAPW + fl), FL)])
      pltpu.sync_copy(lbuf.at[pl.ds(0, FL)],
                      lds_hbm.at[pl.ds(_mo(w * CAPW + fl), FL)])

      def shift(i, _):
        ebuf[pl.ds(_mo(i * 16, 16), 16)] = ebuf[pl.ds(_mo(i * 16 + FL, 16), 16)]
        lbuf[pl.ds(_mo(i * 16, 16), 16)] = lbuf[pl.ds(_mo(i * 16 + FL, 16), 16)]
        return 0

      lax.fori_loop(0, (CBUF - FL) // 16, shift, 0)
      return p - FL, fl + FL

    p, fl = lax.while_loop(flush_cond, flush, (p, fl))
    return p, fl, cnt

  p, fl, cnt = lax.fori_loop(0, N_EDGES // CH, chunk, (0, 0, 0))

  def drain_cond(fc):
    return fc[0] > 0

  def drain(fc):
    p, fl = fc
    pltpu.sync_copy(ebuf.at[pl.ds(0, FL)],
                    eids_hbm.at[pl.ds(_mo(w * CAPW + fl), FL)])
    pltpu.sync_copy(lbuf.at[pl.ds(0, FL)],
                    lds_hbm.at[pl.ds(_mo(w * CAPW + fl), FL)])
    return p - FL, fl + FL

  p, fl = lax.while_loop(drain_cond, drain, (p, fl))

  def bases(t, carry):
    h = histv[pl.ds(_mo(t * 16, 16), 16)]
    hp = jnp.bitwise_and(h + 15, ~15)
    cs = plsc.cumsum(hp)
    bbuf[pl.ds(_mo(t * 16, 16), 16)] = cs - hp + carry
    return carry + jnp.max(cs)

  total = lax.fori_loop(0, NH // 16, bases, 0)
  ok = jnp.int32(0) * jnp.int32(total <= SCAP)
  ngr = total // 16

  def extract(t, _):
    bv = bbuf[pl.ds(_mo(t * 16, 16), 16)]
    for lane in range(16):
      s = jnp.max(jnp.where(iota == lane, bv, 0))
      sm[t * 16 + lane] = s
      sm[NH + t * 16 + lane] = s
    return 0

  lax.fori_loop(0, NH // 16, extract, 0)

  @pl.when(ok == 1)
  def _sort():
    nch = (cnt + FL - 1) // FL

    def pchunk(c, _):
      pltpu.sync_copy(eids_hbm.at[pl.ds(_mo(w * CAPW + c * FL), FL)],
                      ebuf.at[pl.ds(0, FL)])
      pltpu.sync_copy(lds_hbm.at[pl.ds(_mo(w * CAPW + c * FL), FL)],
                      lbuf.at[pl.ds(0, FL)])

      def pstep(i, _):
        gidx = c * FL + i * 16
        eidv = ebuf[pl.ds(_mo(i * 16, 16), 16)]
        ldv = lbuf[pl.ds(_mo(i * 16, 16), 16)]
        for lane in range(16):
          @pl.when(gidx + lane < cnt)
          def _place():
            ld = jnp.max(jnp.where(iota == lane, ldv, 0))
            pos = sm[ld]
            sm[ld] = pos + 1
            plsc.store_scatter(sbufv, [zero + pos], eidv, mask=iota == lane)
        return 0

      lax.fori_loop(0, FL // 16, pstep, 0)
      return 0

    lax.fori_loop(0, nch, pchunk, 0)

    def padrow(t, _):
      lev = laste[pl.ds(_mo(t * 16, 16), 16)]
      for lane in range(16):
        r = t * 16 + lane
        cur = sm[r]
        nxt = sm[NH + r + 1]
        le = jnp.max(jnp.where(iota == lane, lev, 0))
        plsc.store_scatter(sbufv, [cur + iota], zero + le,
                           mask=iota < (nxt - cur))
        g0 = sm[NH + r] // 16

        def gcond(g):
          return g < nxt // 16

        def gfill(g):
          plsc.store_scatter(glbuf, [zero + g], zero + r, mask=iota == 0)
          return g + 1

        lax.while_loop(gcond, gfill, g0)
      return 0

    lax.fori_loop(0, NPW // 16, padrow, 0)

  for blk in range(SCAP // FL):
    pltpu.sync_copy(sbufv.at[pl.ds(blk * FL, FL)],
                    sort_hbm.at[pl.ds(_mo(w * SCAP + blk * FL), FL)])
  pltpu.sync_copy(glbuf, gl_hbm.at[pl.ds(_mo(w * NG), NG)])

  meta = jnp.where(iota == 0, cnt,
                   jnp.where(iota == 1, ok, jnp.where(iota == 2, ngr, 0)))
  dbuf[pl.ds(0, 16)] = meta
  pltpu.sync_copy(dbuf.at[pl.ds(0, 16)], cnts_hbm.at[pl.ds(_mo(w * 16, 16), 16)])


def _make_sc_gather():
  e_out = jax.ShapeDtypeStruct((N_EDGES, D), jnp.float32)

  @functools.partial(
      pl.kernel,
      out_type=e_out,
      mesh=_mesh,
      compiler_params=pltpu.CompilerParams(needs_layout_passes=False),
      scratch_types=[
          pltpu.VMEM((CG,), jnp.int32),
          pltpu.VMEM((CG,), jnp.int32),
          pltpu.VMEM((CG, D), jnp.float32),
          pltpu.VMEM((CG, D), jnp.float32),
          pltpu.SemaphoreType.DMA,
          pltpu.SemaphoreType.DMA,
      ],
  )
  def sc_gather(a_hbm, b_hbm, src_hbm, dst_hbm, g_hbm,
                dstb, srcb, ar, br, sem1, sem2):
    w = _wid()

    def chunk(c, _):
      off = w * EW + c * CG
      pltpu.sync_copy(dst_hbm.at[pl.ds(_mo(off), CG)], dstb)
      pltpu.sync_copy(src_hbm.at[pl.ds(_mo(off), CG)], srcb)
      d1 = pltpu.async_copy(a_hbm.at[dstb], ar, sem1)
      d2 = pltpu.async_copy(b_hbm.at[srcb], br, sem2)
      d1.wait()
      d2.wait()

      def row(r, _):
        for k in range(D // 16):
          sl = pl.ds(k * 16, 16)
          ar[r, sl] = ar[r, sl] + br[r, sl]
        return 0

      lax.fori_loop(0, CG, row, 0)
      pltpu.sync_copy(ar, g_hbm.at[pl.ds(_mo(off), CG)])
      return 0

    lax.fori_loop(0, EW // CG, chunk, 0)

  return sc_gather


_sc_gather = _make_sc_gather()


def _make_sc_scatter_max():
  out = jax.ShapeDtypeStruct((NP, D), jnp.float32)

  @functools.partial(
      pl.kernel,
      out_type=out,
      mesh=_mesh,
      compiler_params=pltpu.CompilerParams(needs_layout_passes=False),
      scratch_types=[
          pltpu.VMEM((FL,), jnp.int32),
          pltpu.VMEM((FL,), jnp.int32),
          pltpu.VMEM((128, D), jnp.float32),
          pltpu.VMEM((128, D), jnp.float32),
          pltpu.VMEM((128, D), jnp.float32),
          pltpu.VMEM(((NPW + 1) * D,), jnp.float32),
          pltpu.VMEM((CG, D), jnp.float32),
          pltpu.VMEM((16,), jnp.int32),
          pltpu.SMEM((FL // 16,), jnp.int32),
          pltpu.SemaphoreType.DMA,
          pltpu.SemaphoreType.DMA,
          pltpu.SemaphoreType.DMA,
      ],
  )
  def sc_scatter(m_hbm, eids_hbm, lds_hbm, cnts_hbm, sort_hbm, gl_hbm,
                 s_hbm, hn_hbm, eidb, ldb, mb0, mb1, mb2, acc, sbuf, cntb, sm,
                 sem0, sem1, sem2):
    mbufs = (mb0, mb1, mb2)
    sems = (sem0, sem1, sem2)
    mbuf = mb0
    sem = sem0
    w = _wid()
    iota = lax.iota(jnp.int32, 16)

    for q in range(NPW // CG):
      pltpu.sync_copy(s_hbm.at[pl.ds(_mo(w * NPW + q * CG), CG)], sbuf)

      def irow(r, _):
        for k in range(D // 16):
          acc[pl.ds(_mo((q * CG + r) * D + k * 16, 16), 16)] = jnp.maximum(
              sbuf[r, pl.ds(k * 16, 16)], 0.0)
        return 0

      lax.fori_loop(0, CG, irow, 0)

    pltpu.sync_copy(cnts_hbm.at[pl.ds(_mo(w * 16, 16), 16)], cntb)
    mv = cntb[...]
    cnt = jnp.max(jnp.where(iota == 0, mv, 0))
    ok = jnp.max(jnp.where(iota == 1, mv, 0))
    ngr = jnp.max(jnp.where(iota == 2, mv, 0))

    @pl.when(ok == 1)
    def _fast():
      nch = (ngr * 16 + FL - 1) // FL

      def chunk(c, _):
        pltpu.sync_copy(sort_hbm.at[pl.ds(_mo(w * SCAP + c * FL), FL)], eidb)
        glv = gl_hbm.at[pl.ds(_mo(w * NG + c * (FL // 16)), FL // 16)]
        pltpu.sync_copy(glv, ldb.at[pl.ds(0, FL // 16)])

        def gl2sm(t, _):
          gv = ldb[pl.ds(_mo(t * 16, 16), 16)]
          for lane in range(16):
            sm[t * 16 + lane] = jnp.max(jnp.where(iota == lane, gv, 0))
          return 0

        lax.fori_loop(0, FL // 256, gl2sm, 0)

        for b in range(3):
          pltpu.async_copy(
              m_hbm.at[eidb.at[pl.ds(_mo(b * 128), 128)]], mbufs[b], sems[b])

        def sub3(s0, _):
          for par in range(3):
            s = s0 * 3 + par
            mb = mbufs[par]
            pltpu.make_async_copy(
                m_hbm.at[eidb.at[pl.ds(_mo(s * 128), 128)]], mb,
                sems[par]).wait()
            for gg in range(8):
              r = sm[s * 8 + gg]
              rb = r * D
              for k in range(D // 16):
                sl = pl.ds(k * 16, 16)
                vals = [mb[gg * 16 + u, sl] for u in range(16)]
                while len(vals) > 1:
                  vals = [jnp.maximum(vals[2 * i], vals[2 * i + 1])
                          for i in range(len(vals) // 2)]
                asl = pl.ds(_mo(rb + k * 16, 16), 16)
                acc[asl] = jnp.maximum(acc[asl], vals[0])

            @pl.when(s + 3 < FL // 128)
            def _prefetch():
              pltpu.async_copy(
                  m_hbm.at[eidb.at[pl.ds(_mo((s + 3) * 128), 128)]], mb,
                  sems[par])
          return 0

        lax.fori_loop(0, FL // (128 * 3), sub3, 0)

        s = 15
        mb = mbufs[s % 3]
        pltpu.make_async_copy(
            m_hbm.at[eidb.at[pl.ds(_mo(s * 128), 128)]], mb, sems[s % 3]).wait()
        for gg in range(8):
          r = sm[s * 8 + gg]
          rb = r * D
          for k in range(D // 16):
            sl = pl.ds(k * 16, 16)
            vals = [mb[gg * 16 + u, sl] for u in range(16)]
            while len(vals) > 1:
              vals = [jnp.maximum(vals[2 * i], vals[2 * i + 1])
                      for i in range(len(vals) // 2)]
            asl = pl.ds(_mo(rb + k * 16, 16), 16)
            acc[asl] = jnp.maximum(acc[asl], vals[0])
        return 0

      lax.fori_loop(0, nch, chunk, 0)

    @pl.when(ok == 0)
    def _slow():
      nch = (cnt + FL - 1) // FL

      def process_sub(s, mb):
        def edge(j, _):
          ldv = ldb[pl.ds(_mo(s * 128 + jnp.bitwise_and(j, ~15), 16), 16)]
          lane = jnp.bitwise_and(j, 15)
          ld = jnp.max(jnp.where(iota == lane, ldv, 0))
          rb = ld * D
          for k in range(D // 16):
            sl = pl.ds(_mo(rb + k * 16, 16), 16)
            acc[sl] = jnp.maximum(acc[sl], mb[j, pl.ds(k * 16, 16)])
          return 0

        lax.fori_loop(0, 128, edge, 0)

      def chunk(c, _):
        pltpu.sync_copy(eids_hbm.at[pl.ds(_mo(w * CAPW + c * FL), FL)], eidb)
        pltpu.sync_copy(lds_hbm.at[pl.ds(_mo(w * CAPW + c * FL), FL)], ldb)

        for b in range(3):
          pltpu.async_copy(
              m_hbm.at[eidb.at[pl.ds(_mo(b * 128), 128)]], mbufs[b], sems[b])

        def sub3(s0, _):
          for par in range(3):
            s = s0 * 3 + par
            mb = mbufs[par]
            pltpu.make_async_copy(
                m_hbm.at[eidb.at[pl.ds(_mo(s * 128), 128)]], mb,
                sems[par]).wait()
            process_sub(s, mb)

            @pl.when(s + 3 < FL // 128)
            def _prefetch():
              pltpu.async_copy(
                  m_hbm.at[eidb.at[pl.ds(_mo((s + 3) * 128), 128)]], mb,
                  sems[par])
          return 0

        lax.fori_loop(0, FL // (128 * 3), sub3, 0)

        s = 15
        mb = mbufs[s % 3]
        pltpu.make_async_copy(
            m_hbm.at[eidb.at[pl.ds(_mo(s * 128), 128)]], mb, sems[s % 3]).wait()
        process_sub(s, mb)
        return 0

      lax.fori_loop(0, nch, chunk, 0)

    for q in range(NPW // CG):
      def orow(r, _):
        for k in range(D // 16):
          sbuf[r, pl.ds(k * 16, 16)] = acc[pl.ds(_mo((q * CG + r) * D + k * 16, 16), 16)]
        return 0

      lax.fori_loop(0, CG, orow, 0)
      pltpu.sync_copy(sbuf, hn_hbm.at[pl.ds(_mo(w * NPW + q * CG), CG)])

  return sc_scatter


_sc_scatter_max = _make_sc_scatter_max()


@jax.jit
def kernel(x, edge_index, edge_attr, W_enc, b_enc,
           W1_0, b1_0, W2_0, b2_0,
           W1_1, b1_1, W2_1, b2_1,
           W1_2, b1_2, W2_2, b2_2,
           W_dec, b_dec):
  src = edge_index[0]
  dst = edge_index[1]
  eap = jnp.pad(edge_attr, ((0, 0), (0, 5)))

  eids, lds, cnts, sort_e, gl = _sc_bin(dst)

  xp = jnp.pad(x, ((0, NP - N_NODES), (0, 0)))
  h = _linear(xp, W_enc.T, b_enc[None, :], act=True)

  for (W1, b1, W2, b2) in ((W1_0, b1_0, W2_0, b2_0),
                           (W1_1, b1_1, W2_1, b2_1),
                           (W1_2, b1_2, W2_2, b2_2)):
    w1it = W1[:, :H].T
    w1jt = W1[:, H:2 * H].T
    w1et = jnp.pad(W1[:, 2 * H:], ((0, 0), (0, 5))).T
    a, b, s = _node_kernel(h, w1it, w1jt, b1[None, :], W2.T, b2[None, :])
    g = _sc_gather(a, b, src, dst)
    m = _edge_kernel(g, eap, w1et, b1[None, :], W2.T, b2[None, :])
    h = _sc_scatter_max(m, eids, lds, cnts, sort_e, gl, s)

  out = _linear(h, W_dec.T, b_dec[None, :], act=False)
  return out[:N_NODES]
